# baseline (speedup 1.0000x reference)
import jax
import jax.numpy as jnp
from jax import lax
from jax.experimental import pallas as pl
from jax.experimental.pallas import tpu as pltpu
from jax.experimental.pallas import tpu_sc as plsc

B = 1024
F = 13
L = 50
EMB = 128
NV = EMB // 16
VOCAB = 1000
NC, NS = 2, 16
NW = NC * NS
SPT = B // NW
FH0, FH1 = 6, 7
RH0 = FH0 * (L + 2)
RH1 = FH1 * (L + 2)
H1C = 384
IDXW = 768
CH0 = ((0, 128), (128, 128), (256, RH0 - 256))
CH1 = ((H1C, 128), (H1C + 128, 128), (H1C + 256, RH1 - 256))


def _fold_reduce(vecs):
    t = vecs[0]
    for v in vecs[1:]:
        t = t + v
    lane = lax.iota(jnp.int32, 16)
    dnums = lax.GatherDimensionNumbers(
        offset_dims=(), collapsed_slice_dims=(0,), start_index_map=(0,))
    for sh in (1, 2, 4, 8):
        perm = jnp.bitwise_xor(lane, sh)
        t = t + lax.gather(t, perm[:, None], dnums, slice_sizes=(1,),
                           mode=lax.GatherScatterMode.PROMISE_IN_BOUNDS)
    return t


def _sc_body(emb_hbm, idx_hbm, ctx_hbm, w39_hbm, wctx_hbm, lrb_hbm,
             out_hbm,
             bufa_v, bufb_v, idx_v, ctx_v, w39_v, wctx_v, lrb_v, out_v,
             sema, semb):
    wid = lax.axis_index("s") * NC + lax.axis_index("c")
    base = wid * SPT

    pltpu.sync_copy(idx_hbm.at[pl.ds(base, SPT)], idx_v)
    pltpu.sync_copy(ctx_hbm.at[pl.ds(base, SPT)], ctx_v)
    pltpu.sync_copy(w39_hbm, w39_v)
    pltpu.sync_copy(wctx_hbm, wctx_v)
    pltpu.sync_copy(lrb_hbm, lrb_v)

    zero = jnp.zeros((16,), jnp.float32)

    def issue(i, buf, sem, chunks):
        off0 = chunks[0][0]
        for off, ln in chunks:
            pltpu.async_copy(emb_hbm.at[idx_v.at[i, pl.ds(off, ln)]],
                             buf.at[pl.ds(off - off0, ln)], sem)

    def drain(buf, sem, chunks):
        off0 = chunks[0][0]
        for off, ln in chunks:
            pltpu.make_async_copy(emb_hbm.at[idx_v.at[0, pl.ds(off, ln)]],
                                  buf.at[pl.ds(off - off0, ln)], sem).wait()

    def compute_half(buf, f0, nf, carry):
        def rd(r):
            return [buf[r, pl.ds(j * 16, 16)] for j in range(NV)]

        def field_body(lf, c):
            s, nrm, wd = c

            def u_body(l5, uacc):
                out = list(uacc)
                for k in range(5):
                    r = rd(2 * nf + lf * L + l5 * 5 + k)
                    out = [a + b for a, b in zip(out, r)]
                return out

            uacc = lax.fori_loop(0, L // 5, u_body, [zero] * NV)
            rt = rd(lf)
            rp = rd(nf + lf)
            f = f0 + lf
            s2, nrm2, wd2 = [], [], []
            for j in range(NV):
                u = uacc[j] * (1.0 / L)
                wt = w39_v[f, pl.ds(j * 16, 16)]
                wu = w39_v[F + f, pl.ds(j * 16, 16)]
                wp = w39_v[2 * F + f, pl.ds(j * 16, 16)]
                s2.append(s[j] + rt[j] + rp[j] + u)
                nrm2.append(nrm[j] + rt[j] * rt[j] + rp[j] * rp[j] + u * u)
                wd2.append(wd[j] + rt[j] * wt + rp[j] * wp + u * wu)
            return (s2, nrm2, wd2)

        return lax.fori_loop(0, nf, field_body, carry)

    issue(0, bufa_v, sema, CH0)

    def sample_body(i, carry):
        issue(i, bufb_v, semb, CH1)
        drain(bufa_v, sema, CH0)
        init = ([zero] * NV, [zero] * NV, [zero] * NV)
        acc = compute_half(bufa_v, 0, FH0, init)
        issue(jnp.minimum(i + 1, SPT - 1), bufa_v, sema, CH0)
        drain(bufb_v, semb, CH1)
        s, nrm, wd = compute_half(bufb_v, FH0, FH1, acc)

        for j in range(4):
            wd[j] = wd[j] + ctx_v[i, pl.ds(j * 16, 16)] * wctx_v[pl.ds(j * 16, 16)]

        sq = [v * v for v in s]
        square_sum = _fold_reduce(sq)
        sum_square = _fold_reduce(nrm)
        lr_dot = _fold_reduce(wd) + lrb_v[pl.ds(0, 16)]

        cross = 1.0 / (1.0 + jnp.exp((sum_square - square_sum) * 0.5))
        lr = 1.0 / (1.0 + jnp.exp(-lr_dot))
        out_v[i] = (cross + lr) * 0.5
        return carry

    lax.fori_loop(0, SPT, sample_body, 0)
    drain(bufa_v, sema, CH0)
    pltpu.sync_copy(out_v, out_hbm.at[pl.ds(base, SPT)])


@jax.jit
def kernel(target_ad, ubs_feature, profile_feature, context_feature,
           item_emb, profile_emb, lr_W, lr_b):
    ta = target_ad.astype(jnp.int32)
    ub = ubs_feature.astype(jnp.int32)
    pf = profile_feature.astype(jnp.int32)
    foff = jnp.arange(F, dtype=jnp.int32) * VOCAB

    emb_all = jnp.concatenate(
        [item_emb.reshape(F * VOCAB, EMB), profile_emb.reshape(F * VOCAB, EMB)],
        axis=0)

    idx_t = ta + foff[None, :]
    idx_p = pf + foff[None, :] + F * VOCAB
    idx_u = (ub.transpose(0, 2, 1) + foff[None, :, None]).reshape(B, F * L)
    idx = jnp.concatenate(
        [idx_t[:, :FH0], idx_p[:, :FH0], idx_u[:, : FH0 * L],
         jnp.zeros((B, H1C - RH0), jnp.int32),
         idx_t[:, FH0:], idx_p[:, FH0:], idx_u[:, FH0 * L:],
         jnp.zeros((B, IDXW - H1C - RH1), jnp.int32)],
        axis=1)

    w39 = lr_W[: 3 * F * EMB, 0].reshape(3 * F, EMB)
    wctx = lr_W[3 * F * EMB:, 0]
    lrb = jnp.broadcast_to(lr_b.astype(jnp.float32), (16,))

    mesh = plsc.VectorSubcoreMesh(core_axis_name="c", subcore_axis_name="s",
                                  num_cores=NC, num_subcores=NS)
    out16 = pl.kernel(
        _sc_body,
        out_type=jax.ShapeDtypeStruct((B, 16), jnp.float32),
        mesh=mesh,
        scratch_types=[
            pltpu.VMEM((RH0, EMB), jnp.float32),
            pltpu.VMEM((RH1, EMB), jnp.float32),
            pltpu.VMEM((SPT, IDXW), jnp.int32),
            pltpu.VMEM((SPT, 64), jnp.float32),
            pltpu.VMEM((3 * F, EMB), jnp.float32),
            pltpu.VMEM((64,), jnp.float32),
            pltpu.VMEM((16,), jnp.float32),
            pltpu.VMEM((SPT, 16), jnp.float32),
            pltpu.SemaphoreType.DMA,
            pltpu.SemaphoreType.DMA,
        ],
    )(emb_all, idx, context_feature.astype(jnp.float32), w39, wctx, lrb)

    return out16[:, :1]


# speedup vs baseline: 1.0523x; 1.0523x over previous
import jax
import jax.numpy as jnp
from jax import lax
from jax.experimental import pallas as pl
from jax.experimental.pallas import tpu as pltpu
from jax.experimental.pallas import tpu_sc as plsc

B = 1024
F = 13
L = 50
EMB = 128
NV = EMB // 16
VOCAB = 1000
NC, NS = 2, 16
NW = NC * NS
SPT = B // NW
FH0, FH1 = 6, 7
IDXW = 904

IT0 = ((0, 8, 128), (128, 136, 128), (256, 264, 50))
PR0 = (384, 0, FH0)
RB0 = 8 + FH0 * (L + 1)
IT1 = ((512, 8, 128), (640, 136, 128), (768, 264, 101))
PR1 = (896, 0, FH1)
RB1 = 8 + FH1 * (L + 1)


def _fold_reduce(vecs):
    t = vecs[0]
    for v in vecs[1:]:
        t = t + v
    lane = lax.iota(jnp.int32, 16)
    dnums = lax.GatherDimensionNumbers(
        offset_dims=(), collapsed_slice_dims=(0,), start_index_map=(0,))
    for sh in (1, 2, 4, 8):
        perm = jnp.bitwise_xor(lane, sh)
        t = t + lax.gather(t, perm[:, None], dnums, slice_sizes=(1,),
                           mode=lax.GatherScatterMode.PROMISE_IN_BOUNDS)
    return t


def _sc_body(itm_hbm, prf_hbm, idx_hbm, ctx_hbm, w40_hbm,
             out_hbm,
             bufa_v, bufb_v, idx_v, ctxout_v, w40_v,
             sema, semb):
    wid = lax.axis_index("s") * NC + lax.axis_index("c")
    base = wid * SPT

    pltpu.sync_copy(idx_hbm.at[pl.ds(base, SPT)], idx_v)
    pltpu.sync_copy(ctx_hbm.at[pl.ds(base, SPT)], ctxout_v)
    pltpu.sync_copy(w40_hbm, w40_v)

    zero = jnp.zeros((16,), jnp.float32)

    def plan(i, buf, sem, items, prof):
        cps = []
        for col, dst, ln in items:
            cps.append(pltpu.make_async_copy(
                itm_hbm.at[idx_v.at[i, pl.ds(col, ln)]],
                buf.at[pl.ds(dst, ln)], sem))
        col, dst, ln = prof
        cps.append(pltpu.make_async_copy(
            prf_hbm.at[idx_v.at[i, pl.ds(col, ln)]],
            buf.at[pl.ds(dst, ln)], sem))
        return cps

    def issue(i, buf, sem, items, prof):
        for cp in plan(i, buf, sem, items, prof):
            cp.start()

    def drain(buf, sem, items, prof):
        for cp in plan(0, buf, sem, items, prof):
            cp.wait()

    def compute_half(buf, f0, nf, carry):
        def rd(r):
            return [buf[r, pl.ds(j * 16, 16)] for j in range(NV)]

        def field_body(lf, c):
            s, nrm, wd = c

            def u_body(l5, uacc):
                out = list(uacc)
                for k in range(5):
                    r = rd(8 + nf + lf * L + l5 * 5 + k)
                    out = [a + b for a, b in zip(out, r)]
                return out

            uacc = lax.fori_loop(0, L // 5, u_body, [zero] * NV)
            rt = rd(8 + lf)
            rp = rd(lf)
            f = f0 + lf
            s2, nrm2, wd2 = [], [], []
            for j in range(NV):
                u = uacc[j] * (1.0 / L)
                wt = w40_v[f, pl.ds(j * 16, 16)]
                wu = w40_v[F + f, pl.ds(j * 16, 16)]
                wp = w40_v[2 * F + f, pl.ds(j * 16, 16)]
                s2.append(s[j] + rt[j] + rp[j] + u)
                nrm2.append(nrm[j] + rt[j] * rt[j] + rp[j] * rp[j] + u * u)
                wd2.append(wd[j] + rt[j] * wt + rp[j] * wp + u * wu)
            return (s2, nrm2, wd2)

        return lax.fori_loop(0, nf, field_body, carry)

    issue(0, bufa_v, sema, IT0, PR0)

    def sample_body(i, carry):
        issue(i, bufb_v, semb, IT1, PR1)
        drain(bufa_v, sema, IT0, PR0)
        init = ([zero] * NV, [zero] * NV, [zero] * NV)
        acc = compute_half(bufa_v, 0, FH0, init)
        issue(jnp.minimum(i + 1, SPT - 1), bufa_v, sema, IT0, PR0)
        drain(bufb_v, semb, IT1, PR1)
        s, nrm, wd = compute_half(bufb_v, FH0, FH1, acc)

        for j in range(4):
            wd[j] = wd[j] + ctxout_v[i, pl.ds(j * 16, 16)] * w40_v[3 * F, pl.ds(j * 16, 16)]

        sq = [v * v for v in s]
        square_sum = _fold_reduce(sq)
        sum_square = _fold_reduce(nrm)
        lr_dot = _fold_reduce(wd) + w40_v[3 * F, pl.ds(64, 16)]

        cross = 1.0 / (1.0 + jnp.exp((sum_square - square_sum) * 0.5))
        lr = 1.0 / (1.0 + jnp.exp(-lr_dot))
        ctxout_v[i, pl.ds(64, 16)] = (cross + lr) * 0.5
        return carry

    lax.fori_loop(0, SPT, sample_body, 0)
    drain(bufa_v, sema, IT0, PR0)
    pltpu.sync_copy(ctxout_v, out_hbm.at[pl.ds(base, SPT)])


@jax.jit
def kernel(target_ad, ubs_feature, profile_feature, context_feature,
           item_emb, profile_emb, lr_W, lr_b):
    ta = target_ad.astype(jnp.int32)
    ub = ubs_feature.astype(jnp.int32)
    pf = profile_feature.astype(jnp.int32)
    foff = jnp.arange(F, dtype=jnp.int32) * VOCAB

    itm = item_emb.reshape(F * VOCAB, EMB)
    prf = profile_emb.reshape(F * VOCAB, EMB)

    idx_t = ta + foff[None, :]
    idx_p = pf + foff[None, :]
    idx_u = (ub.transpose(0, 2, 1) + foff[None, :, None]).reshape(B, F * L)

    def zpad(n):
        return jnp.zeros((B, n), jnp.int32)

    ctx128 = jnp.concatenate(
        [context_feature.astype(jnp.float32),
         jnp.zeros((B, EMB - 64), jnp.float32)], axis=1)

    idx = jnp.concatenate(
        [idx_t[:, :FH0], idx_u[:, : FH0 * L], zpad(78),
         idx_p[:, :FH0], zpad(122),
         idx_t[:, FH0:], idx_u[:, FH0 * L:], zpad(27),
         idx_p[:, FH0:], zpad(1)],
        axis=1)

    w39 = lr_W[: 3 * F * EMB, 0].reshape(3 * F, EMB)
    wrow = jnp.concatenate(
        [lr_W[3 * F * EMB:, 0],
         jnp.broadcast_to(lr_b.astype(jnp.float32), (16,)),
         jnp.zeros((48,), jnp.float32)])
    w40 = jnp.concatenate([w39, wrow[None, :]], axis=0)

    mesh = plsc.VectorSubcoreMesh(core_axis_name="c", subcore_axis_name="s",
                                  num_cores=NC, num_subcores=NS)
    out16 = pl.kernel(
        _sc_body,
        out_type=jax.ShapeDtypeStruct((B, EMB), jnp.float32),
        mesh=mesh,
        scratch_types=[
            pltpu.VMEM((RB0, EMB), jnp.float32),
            pltpu.VMEM((RB1, EMB), jnp.float32),
            pltpu.VMEM((SPT, IDXW), jnp.int32),
            pltpu.VMEM((SPT, EMB), jnp.float32),
            pltpu.VMEM((3 * F + 1, EMB), jnp.float32),
            pltpu.SemaphoreType.DMA,
            pltpu.SemaphoreType.DMA,
        ],
    )(itm, prf, idx, ctx128, w40)

    return out16[:, 64:65]
